# Initial kernel scaffold; baseline (speedup 1.0000x reference)
import jax
import jax.numpy as jnp
from jax.experimental import pallas as pl


def kernel(era_ids, decade_ids, years, visual_styles, audio_styles, era_table, decade_table, visual_table, audio_table, W_y1, b_y1, W_y2, b_y2, W_f1, b_f1, W_f2, b_f2):
    raise NotImplementedError("write your pallas kernel here")


# speedup vs baseline: 6.9281x; 6.9281x over previous
import functools

import jax
import jax.numpy as jnp
from jax.experimental import pallas as pl
from jax.experimental.pallas import tpu as pltpu

_F32 = jnp.float32


def _gelu(x):
    return 0.5 * x * (1.0 + jax.lax.erf(x * 0.7071067811865476))


def _era_kernel(n_rows, c_year, d_year, bblk,
                ids_ref, yr_ref, T_ref, Wf1_ref, Wy1_ref, by1_ref, Wy2_ref,
                by2_ref, bf1_ref, Wf2_ref, bf2_ref, out_ref,
                cat_s, wy_s, bf_s):
    @pl.when(pl.program_id(0) == 0)
    def _precompute():
        cat_s[...] = jnp.dot(T_ref[...], Wf1_ref[...],
                             preferred_element_type=_F32)
        w_year = Wf1_ref[c_year:c_year + d_year, :]
        wy_s[...] = jnp.dot(Wy2_ref[...], w_year, preferred_element_type=_F32)
        bf_s[...] = bf1_ref[...] + jnp.dot(by2_ref[...], w_year,
                                           preferred_element_type=_F32)

    ids = ids_ref[...]
    iot = jax.lax.broadcasted_iota(jnp.int32, (bblk, n_rows), 1)
    oh = ((iot == ids[:, 0:1]).astype(_F32)
          + (iot == ids[:, 1:2]).astype(_F32)
          + (iot == ids[:, 2:3]).astype(_F32)
          + (iot == ids[:, 3:4]).astype(_F32))

    yn = (yr_ref[...].astype(_F32) - 1920.0) / 110.0
    y1 = _gelu(yn * Wy1_ref[...] + by1_ref[...])

    acc = jnp.dot(oh, cat_s[...], preferred_element_type=_F32)
    acc = acc + jnp.dot(y1, wy_s[...], preferred_element_type=_F32)
    acc = acc + bf_s[...]
    h = _gelu(acc)
    out_ref[...] = jnp.dot(h, Wf2_ref[...],
                           preferred_element_type=_F32) + bf2_ref[...]


def kernel(era_ids, decade_ids, years, visual_styles, audio_styles,
           era_table, decade_table, visual_table, audio_table,
           W_y1, b_y1, W_y2, b_y2, W_f1, b_f1, W_f2, b_f2):
    B = era_ids.shape[0]
    n_era, d_era = era_table.shape
    n_dec, d_dec = decade_table.shape
    n_vis, d_vis = visual_table.shape
    n_aud, d_aud = audio_table.shape
    d_year = W_y1.shape[1]
    d_in = d_era + d_dec + d_year + d_vis + d_aud
    H = W_f2.shape[1]

    r_dec = n_era
    r_vis = r_dec + n_dec
    r_aud = r_vis + n_vis
    n_rows = -(-(r_aud + n_aud) // 128) * 128
    c_dec = d_era
    c_year = c_dec + d_dec
    c_vis = c_year + d_year
    c_aud = c_vis + d_vis

    T = jnp.zeros((n_rows, d_in), dtype=_F32)
    T = T.at[0:n_era, 0:d_era].set(era_table)
    T = T.at[r_dec:r_dec + n_dec, c_dec:c_dec + d_dec].set(decade_table)
    T = T.at[r_vis:r_vis + n_vis, c_vis:c_vis + d_vis].set(visual_table)
    T = T.at[r_aud:r_aud + n_aud, c_aud:c_aud + d_aud].set(audio_table)

    ids = jnp.stack(
        [era_ids.astype(jnp.int32),
         decade_ids.astype(jnp.int32) + r_dec,
         visual_styles.astype(jnp.int32) + r_vis,
         audio_styles.astype(jnp.int32) + r_aud], axis=1)
    yrs = years.astype(jnp.int32).reshape(B, 1)

    bblk = 1024
    grid = (B // bblk,)

    full = lambda shape: pl.BlockSpec(shape, lambda i: (0, 0))
    out = pl.pallas_call(
        functools.partial(_era_kernel, n_rows, c_year, d_year, bblk),
        grid=grid,
        in_specs=[
            pl.BlockSpec((bblk, 4), lambda i: (i, 0)),
            pl.BlockSpec((bblk, 1), lambda i: (i, 0)),
            full((n_rows, d_in)),
            full((d_in, H)),
            full((1, d_year)),
            full((1, d_year)),
            full((d_year, d_year)),
            full((1, d_year)),
            full((1, H)),
            full((H, H)),
            full((1, H)),
        ],
        out_specs=pl.BlockSpec((bblk, H), lambda i: (i, 0)),
        out_shape=jax.ShapeDtypeStruct((B, H), _F32),
        scratch_shapes=[
            pltpu.VMEM((n_rows, H), _F32),
            pltpu.VMEM((d_year, H), _F32),
            pltpu.VMEM((1, H), _F32),
        ],
    )(ids, yrs, T, W_f1, W_y1, b_y1.reshape(1, d_year), W_y2,
      b_y2.reshape(1, d_year), b_f1.reshape(1, H), W_f2, b_f2.reshape(1, H))
    return out
